# Initial kernel scaffold; baseline (speedup 1.0000x reference)
import jax
import jax.numpy as jnp
from jax.experimental import pallas as pl


def kernel(gt_boxes, labels):
    raise NotImplementedError("write your pallas kernel here")


# speedup vs baseline: 10.6164x; 10.6164x over previous
import math

import numpy as np
import jax
import jax.numpy as jnp
from jax.experimental import pallas as pl
from jax.experimental.pallas import tpu as pltpu

_FM_SIZES = [(64, 64), (32, 32), (16, 16), (8, 8), (4, 4), (2, 2)]
_STEPS = [(8, 8), (16, 16), (32, 32), (64, 64), (128, 128), (256, 256)]
_BOX_SIZES = [35.84, 76.8, 153.6, 230.4, 307.2, 384.0]
_ASPECT_RATIOS = [1.0, 2.0, 0.5]
_SCALES = [1.0, 1.2599]
_FG = 0.6
_BG = 0.4
_V0 = 0.1
_V1 = 0.2

_G = 64
_P = 32760
_PP = 32768
_CHUNK = 4096
_NCH = _PP // _CHUNK


def _priors_np():
    out = []
    for i, (fy, fx) in enumerate(_FM_SIZES):
        sy, sx = _STEPS[i]
        base = _BOX_SIZES[i]
        hh, ww = np.meshgrid(np.arange(fy), np.arange(fx), indexing="ij")
        cx = (ww + 0.5) * sx
        cy = (hh + 0.5) * sy
        cxy = np.stack([cx, cy], axis=-1).reshape(-1, 1, 2).astype(np.float32)
        whs = []
        for ar in _ASPECT_RATIOS:
            for sc in _SCALES:
                whs.append((base * sc * math.sqrt(ar), base * sc / math.sqrt(ar)))
        wh = np.asarray(whs, dtype=np.float32).reshape(1, -1, 2)
        a = wh.shape[1]
        ncell = cxy.shape[0]
        b = np.concatenate(
            [np.broadcast_to(cxy, (ncell, a, 2)), np.broadcast_to(wh, (ncell, a, 2))],
            axis=-1,
        )
        out.append(b.reshape(-1, 4).astype(np.float32))
    return np.concatenate(out, axis=0)


def _parr_np():
    d = _priors_np()
    x1y1 = d[:, :2] - d[:, 2:] / 2.0
    x2y2 = d[:, :2] + d[:, 2:] / 2.0
    dxy = np.concatenate([x1y1, x2y2], axis=1).astype(np.float32)
    area = (dxy[:, 2] - dxy[:, 0]) * (dxy[:, 3] - dxy[:, 1])
    parr = np.zeros((16, _PP), dtype=np.float32)
    parr[0, :_P] = dxy[:, 0]
    parr[1, :_P] = dxy[:, 1]
    parr[2, :_P] = dxy[:, 2]
    parr[3, :_P] = dxy[:, 3]
    parr[4, :_P] = area
    parr[5, :_P] = d[:, 0]
    parr[6, :_P] = d[:, 1]
    parr[7, :_P] = d[:, 2]
    parr[8, :_P] = d[:, 3]
    parr[0:4, _P:] = -1.0e6
    parr[4, _P:] = 0.0
    parr[5:7, _P:] = 0.0
    parr[7:9, _P:] = 1.0
    return parr


_PARR = _parr_np()


def _tc_body(gcols_ref, gtab_ref, parr_ref, loc_ref, cls_ref):
    gx1 = gcols_ref[:, 0:1]
    gy1 = gcols_ref[:, 1:2]
    gx2 = gcols_ref[:, 2:3]
    gy2 = gcols_ref[:, 3:4]
    gar = gcols_ref[:, 4:5]
    giota = jax.lax.broadcasted_iota(jnp.int32, (_G, 1), 0)

    rv = jnp.full((_G, 1), -1.0, jnp.float32)
    ri = jnp.zeros((_G, 1), jnp.int32)
    bts = []
    bis = []
    for ci in range(_NCH):
        b = ci * _CHUNK
        px1 = parr_ref[0:1, b:b + _CHUNK]
        py1 = parr_ref[1:2, b:b + _CHUNK]
        px2 = parr_ref[2:3, b:b + _CHUNK]
        py2 = parr_ref[3:4, b:b + _CHUNK]
        pa = parr_ref[4:5, b:b + _CHUNK]
        ltx = jnp.maximum(px1, gx1)
        lty = jnp.maximum(py1, gy1)
        rbx = jnp.minimum(px2, gx2)
        rby = jnp.minimum(py2, gy2)
        wx = jnp.maximum(rbx - ltx, 0.0)
        wy = jnp.maximum(rby - lty, 0.0)
        inter = wx * wy
        den = (pa + gar) - inter + 1e-10
        iou = inter / den
        bt = jnp.max(iou, axis=0, keepdims=True)
        bi = jnp.min(jnp.where(iou == bt, giota, _G), axis=0, keepdims=True)
        bts.append(bt)
        bis.append(bi)
        cm = jnp.max(iou, axis=1, keepdims=True)
        prow = jax.lax.broadcasted_iota(jnp.int32, (1, _CHUNK), 1) + b
        cidx = jnp.min(jnp.where(iou == cm, prow, _PP), axis=1, keepdims=True)
        upd = cm > rv
        rv = jnp.where(upd, cm, rv)
        ri = jnp.where(upd, cidx, ri)

    for ci in range(_NCH):
        b = ci * _CHUNK
        bt = bts[ci]
        bi = bis[ci]
        prow = jax.lax.broadcasted_iota(jnp.int32, (1, _CHUNK), 1) + b
        eqf = prow == ri
        forced = jnp.any(eqf, axis=0, keepdims=True)
        fgi = jnp.max(jnp.where(eqf, giota, -1), axis=0, keepdims=True)
        bt2 = jnp.where(forced, 2.0, bt)
        bi2 = jnp.where(forced, fgi, bi)
        oneh = (giota == bi2).astype(jnp.float32)
        gat = jnp.dot(gtab_ref[:, :], oneh, preferred_element_type=jnp.float32,
                      precision=jax.lax.Precision.HIGHEST)
        bcx = gat[0:1, :]
        bcy = gat[1:2, :]
        bw = gat[2:3, :]
        bh = gat[3:4, :]
        labf = gat[4:5, :]
        pcx = parr_ref[5:6, b:b + _CHUNK]
        pcy = parr_ref[6:7, b:b + _CHUNK]
        pw = parr_ref[7:8, b:b + _CHUNK]
        ph = parr_ref[8:9, b:b + _CHUNK]
        loc_ref[0:1, b:b + _CHUNK] = (bcx - pcx) / pw / _V0
        loc_ref[1:2, b:b + _CHUNK] = (bcy - pcy) / ph / _V0
        loc_ref[2:3, b:b + _CHUNK] = jnp.log(bw / pw) / _V1
        loc_ref[3:4, b:b + _CHUNK] = jnp.log(bh / ph) / _V1
        lab = labf.astype(jnp.int32)
        cls = jnp.where(bt2 < _FG, -1, lab)
        cls = jnp.where(bt2 < _BG, 0, cls)
        cls_ref[0:1, b:b + _CHUNK] = cls


def kernel(gt_boxes, labels):
    gt_boxes = gt_boxes.astype(jnp.float32)
    x1 = gt_boxes[:, 0:1]
    y1 = gt_boxes[:, 1:2]
    x2 = gt_boxes[:, 2:3]
    y2 = gt_boxes[:, 3:4]
    gar = (x2 - x1) * (y2 - y1)
    z = jnp.zeros((_G, 1), jnp.float32)
    gcols = jnp.concatenate([x1, y1, x2, y2, gar, z, z, z], axis=1)
    bcx = (x1 + x2) / 2.0
    bcy = (y1 + y2) / 2.0
    bw = x2 - x1
    bh = y2 - y1
    labf = (labels + 1).astype(jnp.float32)[:, None]
    gtab = jnp.concatenate([bcx, bcy, bw, bh, labf, z, z, z], axis=1).T
    parr = jnp.asarray(_PARR)
    loc_t, cls_t = pl.pallas_call(
        _tc_body,
        out_shape=[
            jax.ShapeDtypeStruct((4, _PP), jnp.float32),
            jax.ShapeDtypeStruct((1, _PP), jnp.int32),
        ],
    )(gcols, gtab, parr)
    loc = loc_t.T[:_P]
    cls = cls_t[0, :_P]
    return (loc, cls)
